# baseline (speedup 1.0000x reference)
import jax
import jax.numpy as jnp
from jax.experimental import pallas as pl

E = 8
K = 2
H = 1024
F = 1408
T = 2048

TM = 256


def _moe_dense_kernel(ids_ref, w_ref, xt_ref, gu_ref, dn_ref, out_ref):
    e = pl.program_id(0)
    t = pl.program_id(1)

    tok = pl.ds(t * TM, TM)
    x_tile = xt_ref[:, tok].astype(jnp.bfloat16)

    gu = gu_ref[0].astype(jnp.bfloat16)
    h = jnp.dot(gu, x_tile, preferred_element_type=jnp.float32)
    gate = h[:F, :]
    up = h[F:, :]
    act = (jax.nn.silu(gate) * up).astype(jnp.bfloat16)

    dn = dn_ref[0].astype(jnp.bfloat16)
    y = jnp.dot(dn, act, preferred_element_type=jnp.float32)

    ids = ids_ref[tok, :]
    w = w_ref[tok, :]
    wte = jnp.sum(jnp.where(ids == e, w, 0.0), axis=1)
    contrib = y * wte[None, :]

    @pl.when(e == 0)
    def _init():
        out_ref[:, tok] = contrib

    @pl.when(e > 0)
    def _acc():
        out_ref[:, tok] += contrib


@jax.jit
def kernel(x, topk_ids, topk_weight, gate_up_weights, down_weights):
    ids = topk_ids.astype(jnp.int32)
    xt = x.T

    grid = (E, T // TM)
    out_t = pl.pallas_call(
        _moe_dense_kernel,
        grid=grid,
        in_specs=[
            pl.BlockSpec((T, K), lambda e, t: (0, 0)),
            pl.BlockSpec((T, K), lambda e, t: (0, 0)),
            pl.BlockSpec((H, T), lambda e, t: (0, 0)),
            pl.BlockSpec((1, 2 * F, H), lambda e, t: (e, 0, 0)),
            pl.BlockSpec((1, H, F), lambda e, t: (e, 0, 0)),
        ],
        out_specs=pl.BlockSpec((H, T), lambda e, t: (0, 0)),
        out_shape=jax.ShapeDtypeStruct((H, T), jnp.float32),
    )(ids, topk_weight, xt, gate_up_weights, down_weights)
    return out_t.T


# speedup vs baseline: 1.1284x; 1.1284x over previous
import jax
import jax.numpy as jnp
from jax.experimental import pallas as pl
from jax.experimental.pallas import tpu as pltpu

E = 8
K = 2
H = 1024
F = 1408
T = 2048

TM = 256
FC = 128
NF = F // FC


def _moe_dense_kernel(ids_ref, w_ref, xt_ref, gu_ref, dn_ref, out_ref, act_ref):
    e = pl.program_id(0)
    f = pl.program_id(1)

    guc = gu_ref[0].astype(jnp.bfloat16)
    gate_w = guc[0]
    up_w = guc[1]

    frows = pl.ds(f * FC, FC)
    for t in range(T // TM):
        tok = pl.ds(t * TM, TM)
        x_tile = xt_ref[:, tok]
        hg = jnp.dot(gate_w, x_tile, preferred_element_type=jnp.float32)
        hu = jnp.dot(up_w, x_tile, preferred_element_type=jnp.float32)
        act_ref[frows, tok] = (jax.nn.silu(hg) * hu).astype(jnp.bfloat16)

    @pl.when(f == NF - 1)
    def _down():
        dnc = dn_ref[0].astype(jnp.bfloat16)
        for t in range(T // TM):
            tok = pl.ds(t * TM, TM)
            y = jnp.dot(dnc, act_ref[:, tok],
                        preferred_element_type=jnp.float32)
            ids = ids_ref[tok, :]
            w = w_ref[tok, :]
            wte = jnp.sum(jnp.where(ids == e, w, 0.0), axis=1)
            contrib = y * wte[None, :]
            if True:
                @pl.when(e == 0)
                def _init():
                    out_ref[:, tok] = contrib

                @pl.when(e > 0)
                def _acc():
                    out_ref[:, tok] += contrib


@jax.jit
def kernel(x, topk_ids, topk_weight, gate_up_weights, down_weights):
    ids = topk_ids.astype(jnp.int32)
    xt = x.T.astype(jnp.bfloat16)
    gu4 = gate_up_weights.reshape(E, 2, F, H)

    grid = (E, NF)
    out_t = pl.pallas_call(
        _moe_dense_kernel,
        grid=grid,
        in_specs=[
            pl.BlockSpec((T, K), lambda e, f: (0, 0)),
            pl.BlockSpec((T, K), lambda e, f: (0, 0)),
            pl.BlockSpec((H, T), lambda e, f: (0, 0)),
            pl.BlockSpec((1, 2, FC, H), lambda e, f: (e, 0, f, 0)),
            pl.BlockSpec((1, H, F), lambda e, f: (e, 0, 0)),
        ],
        out_specs=pl.BlockSpec((H, T), lambda e, f: (0, 0)),
        out_shape=jax.ShapeDtypeStruct((H, T), jnp.float32),
        scratch_shapes=[pltpu.VMEM((F, T), jnp.bfloat16)],
    )(ids, topk_weight, xt, gu4, down_weights)
    return out_t.T
